# baseline (speedup 1.0000x reference)
import functools

import jax
import jax.numpy as jnp
from jax import lax
from jax.experimental import pallas as pl
from jax.experimental.pallas import tpu as pltpu
from jax.experimental.pallas import tpu_sc as plsc

N = 10000
E = 320000
HID = 128
TAIL = 16

NC, NS = 2, 16
NW = NC * NS
EPW = E // NW
KCH = 80
NCH = EPW // KCH
NP = 10240
SPT = NP // NS
NT = 4
TR = 320

F32 = jnp.float32
BF16 = jnp.bfloat16


def _silu(x):
    return x * (1.0 / (1.0 + jnp.exp(-x)))


def _node0_body(h, cp, wemb, bemb, w1a, be1, w1b, hh_o, ta_o, tb_o):
    hh = jnp.dot(h[...], wemb[...], preferred_element_type=F32) + bemb[...]
    hh_o[...] = hh
    ta_o[...] = jnp.dot(hh, w1a[...], preferred_element_type=F32) + be1[...]
    tb_o[...] = jnp.dot(hh, w1b[...], preferred_element_type=F32)


def _node_body(h, cp, vp, p0, p1, p2, p3, p4, p5, p6, p7, pt, wv1, bv1,
               wv2, bv2, wn1a, wn1b, bn1, wn2, bn2, w1a, be1, w1b,
               hn_o, cn_o, ta_o, tb_o):
    hv = h[...]
    aggm = ((p0[...] + p1[...]) + (p2[...] + p3[...])
            + (p4[...] + p5[...]) + (p6[...] + p7[...]))
    tl = pt[...]
    cnt = jnp.maximum(tl[:, 3:4], 1.0)
    lane = lax.broadcasted_iota(jnp.int32, (1, TAIL), 1)
    mask3 = (lane < 3).astype(F32)
    tl16 = jnp.concatenate(
        [tl, jnp.zeros((tl.shape[0], TAIL - NT), F32)], axis=1)
    sv = (jnp.dot(_silu(jnp.dot(hv, wv1[...], preferred_element_type=F32)
                        + bv1[...]), wv2[...], preferred_element_type=F32)
          + bv2[...])
    cn = cp[...] + (tl16 * mask3) / cnt + sv * vp[...]
    cn_o[...] = cn
    t = _silu(jnp.dot(hv, wn1a[...], preferred_element_type=F32)
              + jnp.dot(aggm, wn1b[...], preferred_element_type=F32)
              + bn1[...])
    hn = hv + jnp.dot(t, wn2[...], preferred_element_type=F32) + bn2[...]
    hn_o[...] = hn
    ta_o[...] = jnp.dot(hn, w1a[...], preferred_element_type=F32) + be1[...]
    tb_o[...] = jnp.dot(hn, w1b[...], preferred_element_type=F32)


def _node_last_body(h, cp, vp, pt, wv1, bv1, wv2, bv2, cn_o):
    hv = h[...]
    tl = pt[...]
    cnt = jnp.maximum(tl[:, 3:4], 1.0)
    lane = lax.broadcasted_iota(jnp.int32, (1, TAIL), 1)
    mask3 = (lane < 3).astype(F32)
    tl16 = jnp.concatenate(
        [tl, jnp.zeros((tl.shape[0], TAIL - NT), F32)], axis=1)
    sv = (jnp.dot(_silu(jnp.dot(hv, wv1[...], preferred_element_type=F32)
                        + bv1[...]), wv2[...], preferred_element_type=F32)
          + bv2[...])
    cn_o[...] = cp[...] + (tl16 * mask3) / cnt + sv * vp[...]


def _edge_body(g, es, ea, wr, w1e, we2, be2, wc1, bc1, wc2, m_o, s_o):
    radial = es[:, 0:1]
    e1 = (g[...] + radial * wr[...]
          + jnp.dot(ea[...], w1e[...], preferred_element_type=F32))
    m = _silu(jnp.dot(_silu(e1).astype(BF16), we2[...].astype(BF16),
                      preferred_element_type=F32) + be2[...])
    m_o[...] = m
    cm = _silu(jnp.dot(m.astype(BF16), wc1[...].astype(BF16),
                       preferred_element_type=F32) + bc1[...])
    s = jnp.dot(cm, wc2[...], preferred_element_type=F32)
    s_o[...] = jnp.concatenate(
        [s, es[:, 1:4], jnp.zeros((s.shape[0], TAIL - 4), F32)], axis=1)


def _edge_body_nom(g, es, ea, wr, w1e, we2, be2, wc1, bc1, wc2, s_o):
    radial = es[:, 0:1]
    e1 = (g[...] + radial * wr[...]
          + jnp.dot(ea[...], w1e[...], preferred_element_type=F32))
    m = _silu(jnp.dot(_silu(e1).astype(BF16), we2[...].astype(BF16),
                      preferred_element_type=F32) + be2[...])
    cm = _silu(jnp.dot(m.astype(BF16), wc1[...].astype(BF16),
                       preferred_element_type=F32) + bc1[...])
    s = jnp.dot(cm, wc2[...], preferred_element_type=F32)
    s_o[...] = jnp.concatenate(
        [s, es[:, 1:4], jnp.zeros((s.shape[0], TAIL - 4), F32)], axis=1)


def _reduce_body(pa, pb, pc, pd, out):
    out[...] = ((jnp.sum(pa[...], axis=0) + jnp.sum(pb[...], axis=0))
                + (jnp.sum(pc[...], axis=0) + jnp.sum(pd[...], axis=0)))


def _make_gather_body(epw, nch):
    pairs = (nch - 1) // 2
    odd = nch % 2 == 1

    def body_fn(ta, tb, row, col, cx, cy, cz, g_o, es_o,
                rowi, coli, bufa0, bufb0, bufa1, bufb1, gsum, esb,
                cxv, cyv, czv, sga0, sgb0, sga1, sgb1, semw0, semw1):
        wid = lax.axis_index("s") * NC + lax.axis_index("c")
        base = wid * epw
        pltpu.sync_copy(row.at[pl.ds(base, epw)], rowi)
        pltpu.sync_copy(col.at[pl.ds(base, epw)], coli)
        pltpu.sync_copy(cx, cxv)
        pltpu.sync_copy(cy, cyv)
        pltpu.sync_copy(cz, czv)
        iota = lax.iota(jnp.int32, 16)

        def issue_g(c, bufa, bufb, sa, sb):
            pltpu.async_copy(ta.at[rowi.at[pl.ds(c * KCH, KCH)]], bufa, sa)
            pltpu.async_copy(tb.at[coli.at[pl.ds(c * KCH, KCH)]], bufb, sb)

        def wait_g(c, bufa, bufb, sa, sb):
            pltpu.make_async_copy(ta.at[rowi.at[pl.ds(c * KCH, KCH)]],
                                  bufa, sa).wait()
            pltpu.make_async_copy(tb.at[coli.at[pl.ds(c * KCH, KCH)]],
                                  bufb, sb).wait()

        def compute_es(c):
            for j in range(KCH // 16):
                rv = rowi[pl.ds(c * KCH + j * 16, 16)]
                cv = coli[pl.ds(c * KCH + j * 16, 16)]
                dx = (plsc.load_gather(cxv, [rv])
                      - plsc.load_gather(cxv, [cv]))
                dy = (plsc.load_gather(cyv, [rv])
                      - plsc.load_gather(cyv, [cv]))
                dz = (plsc.load_gather(czv, [rv])
                      - plsc.load_gather(czv, [cv]))
                r = dx * dx + dy * dy + dz * dz
                ri = iota + (j * 16)
                plsc.store_scatter(esb, [ri, iota * 0], r)
                plsc.store_scatter(esb, [ri, iota * 0 + 1], dx)
                plsc.store_scatter(esb, [ri, iota * 0 + 2], dy)
                plsc.store_scatter(esb, [ri, iota * 0 + 3], dz)

        def add_rows(bufa, bufb):
            def radd(r, _):
                for j in range(HID // 16):
                    gsum[r, pl.ds(j * 16, 16)] = (
                        bufa[r, pl.ds(j * 16, 16)]
                        + bufb[r, pl.ds(j * 16, 16)])
                return 0

            lax.fori_loop(0, KCH, radd, 0)

        def drain(c, semw):
            off = base + c * KCH
            wg = pltpu.async_copy(gsum, g_o.at[pl.ds(off, KCH)], semw)
            we = pltpu.async_copy(esb, es_o.at[pl.ds(off, KCH)], semw)
            wg.wait()
            we.wait()

        def step(c, bufa, bufb, sa, sb, semw):
            compute_es(c)
            wait_g(c, bufa, bufb, sa, sb)
            add_rows(bufa, bufb)
            drain(c, semw)

        issue_g(0, bufa0, bufb0, sga0, sgb0)
        issue_g(1, bufa1, bufb1, sga1, sgb1)

        def body(i, _):
            c0 = 2 * i
            c1 = c0 + 1
            step(c0, bufa0, bufb0, sga0, sgb0, semw0)
            issue_g(c0 + 2, bufa0, bufb0, sga0, sgb0)
            step(c1, bufa1, bufb1, sga1, sgb1, semw1)
            if odd:
                @pl.when(i < pairs - 1)
                def _():
                    issue_g(c1 + 2, bufa1, bufb1, sga1, sgb1)
            else:
                issue_g(c1 + 2, bufa1, bufb1, sga1, sgb1)
            return 0

        lax.fori_loop(0, pairs, body, 0)
        if odd:
            step(nch - 1, bufa0, bufb0, sga0, sgb0, semw0)
        else:
            step(nch - 2, bufa0, bufb0, sga0, sgb0, semw0)
            step(nch - 1, bufa1, bufb1, sga1, sgb1, semw1)

    return body_fn


def _make_scatter_m_body(epw, nch):
    pairs = (nch - 1) // 2
    odd = nch % 2 == 1

    def body_fn(m, row3, zrows, pm_o, idr, mb0, mb1, accm,
                sl0, sl1, sa0, sa1):
        cid = lax.axis_index("c")
        sid = lax.axis_index("s")
        wid = sid * NC + cid
        base = wid * epw

        def issue_l(c, mb, sl):
            pltpu.async_copy(m.at[pl.ds(base + c * KCH, KCH)], mb, sl)

        def wait_l(c, mb, sl):
            pltpu.make_async_copy(m.at[pl.ds(base + c * KCH, KCH)], mb,
                                  sl).wait()

        issue_l(0, mb0, sl0)
        issue_l(1, mb1, sl1)
        pltpu.sync_copy(row3.at[wid], idr)
        pltpu.sync_copy(zrows, accm.at[pl.ds(sid * SPT, SPT)])
        plsc.subcore_barrier()

        def sadd(c, mb, sa):
            pltpu.async_copy(mb, accm.at[idr.at[c]], sa, add=True).wait()

        def step(c, mb, sl, sa):
            wait_l(c, mb, sl)
            sadd(c, mb, sa)

        def body(i, _):
            c0 = 2 * i
            c1 = c0 + 1
            step(c0, mb0, sl0, sa0)
            issue_l(c0 + 2, mb0, sl0)
            step(c1, mb1, sl1, sa1)
            if odd:
                @pl.when(i < pairs - 1)
                def _():
                    issue_l(c1 + 2, mb1, sl1)
            else:
                issue_l(c1 + 2, mb1, sl1)
            return 0

        lax.fori_loop(0, pairs, body, 0)
        if odd:
            step(nch - 1, mb0, sl0, sa0)
        else:
            step(nch - 2, mb0, sl0, sa0)
            step(nch - 1, mb1, sl1, sa1)
        plsc.subcore_barrier()
        pltpu.sync_copy(accm.at[pl.ds(sid * SPT, SPT)],
                        pm_o.at[cid, pl.ds(sid * SPT, SPT)])

    return body_fn


def _make_scatter_t_body(epw, nch):
    pairs = (nch - 1) // 2
    odd = nch % 2 == 1

    def body_fn(s16, row3, pt_o, idr, sb0, sb1, acct, ss0, ss1):
        cid = lax.axis_index("c")
        sid = lax.axis_index("s")
        wid = sid * NC + cid
        base = wid * epw

        def issue_l(c, sb, ss):
            pltpu.async_copy(s16.at[pl.ds(base + c * KCH, KCH)], sb, ss)

        def wait_l(c, sb, ss):
            pltpu.make_async_copy(s16.at[pl.ds(base + c * KCH, KCH)], sb,
                                  ss).wait()

        issue_l(0, sb0, ss0)
        issue_l(1, sb1, ss1)
        pltpu.sync_copy(row3.at[wid], idr)
        zero16 = jnp.zeros((16,), F32)

        def zbody(i, _):
            for j in range(HID // 16):
                acct[i, pl.ds(j * 16, 16)] = zero16
            return 0

        lax.fori_loop(0, TR, zbody, 0)
        iota = lax.iota(jnp.int32, 16)
        one16 = zero16 + 1.0

        def process(c, sb):
            for j in range(KCH // 16):
                rv = idr[c, pl.ds(j * 16, 16)]
                ri = iota + (j * 16)
                sv = plsc.load_gather(sb, [ri, iota * 0])
                dx = plsc.load_gather(sb, [ri, iota * 0 + 1])
                dy = plsc.load_gather(sb, [ri, iota * 0 + 2])
                dz = plsc.load_gather(sb, [ri, iota * 0 + 3])
                fb = rv * NT
                for k, v in ((0, dx * sv), (1, dy * sv), (2, dz * sv),
                             (3, one16)):
                    fk = fb + k
                    plsc.addupdate_scatter(
                        acct, [lax.shift_right_logical(fk, 7), fk & 127], v)

        def step(c, sb, ss):
            wait_l(c, sb, ss)
            process(c, sb)

        def body(i, _):
            c0 = 2 * i
            c1 = c0 + 1
            step(c0, sb0, ss0)
            issue_l(c0 + 2, sb0, ss0)
            step(c1, sb1, ss1)
            if odd:
                @pl.when(i < pairs - 1)
                def _():
                    issue_l(c1 + 2, sb1, ss1)
            else:
                issue_l(c1 + 2, sb1, ss1)
            return 0

        lax.fori_loop(0, pairs, body, 0)
        if odd:
            step(nch - 1, sb0, ss0)
        else:
            step(nch - 2, sb0, ss0)
            step(nch - 1, sb1, ss1)
        pltpu.sync_copy(acct, pt_o.at[wid])

    return body_fn


@functools.cache
def _sc_kernels(epw, nch):
    eh = epw * NW
    mesh = plsc.VectorSubcoreMesh(core_axis_name="c", subcore_axis_name="s",
                                  num_cores=NC, num_subcores=NS)
    cparams = pltpu.CompilerParams(needs_layout_passes=False)
    gather = functools.partial(
        pl.kernel,
        compiler_params=cparams,
        out_type=(jax.ShapeDtypeStruct((eh, HID), F32),
                  jax.ShapeDtypeStruct((eh, TAIL), F32)),
        mesh=mesh,
        scratch_types=[
            pltpu.VMEM((epw,), jnp.int32),
            pltpu.VMEM((epw,), jnp.int32),
            pltpu.VMEM((KCH, HID), F32),
            pltpu.VMEM((KCH, HID), F32),
            pltpu.VMEM((KCH, HID), F32),
            pltpu.VMEM((KCH, HID), F32),
            pltpu.VMEM((KCH, HID), F32),
            pltpu.VMEM((KCH, TAIL), F32),
            pltpu.VMEM((N,), F32),
            pltpu.VMEM((N,), F32),
            pltpu.VMEM((N,), F32),
            pltpu.SemaphoreType.DMA,
            pltpu.SemaphoreType.DMA,
            pltpu.SemaphoreType.DMA,
            pltpu.SemaphoreType.DMA,
            pltpu.SemaphoreType.DMA,
            pltpu.SemaphoreType.DMA,
        ],
    )(_make_gather_body(epw, nch))
    scatter_m = functools.partial(
        pl.kernel,
        compiler_params=cparams,
        out_type=jax.ShapeDtypeStruct((NC, NP, HID), F32),
        mesh=mesh,
        scratch_types=[
            pltpu.VMEM((nch, KCH), jnp.int32),
            pltpu.VMEM((KCH, HID), F32),
            pltpu.VMEM((KCH, HID), F32),
            pltpu.VMEM_SHARED((NP, HID), F32),
            pltpu.SemaphoreType.DMA,
            pltpu.SemaphoreType.DMA,
            pltpu.SemaphoreType.DMA,
            pltpu.SemaphoreType.DMA,
        ],
    )(_make_scatter_m_body(epw, nch))
    scatter_t = functools.partial(
        pl.kernel,
        compiler_params=cparams,
        out_type=jax.ShapeDtypeStruct((NW, TR, HID), F32),
        mesh=mesh,
        scratch_types=[
            pltpu.VMEM((nch, KCH), jnp.int32),
            pltpu.VMEM((KCH, TAIL), F32),
            pltpu.VMEM((KCH, TAIL), F32),
            pltpu.VMEM((TR, HID), F32),
            pltpu.SemaphoreType.DMA,
            pltpu.SemaphoreType.DMA,
        ],
    )(_make_scatter_t_body(epw, nch))
    return gather, scatter_m, scatter_t


PART_NCH = (32, 31, 31, 31)
PART_E = tuple(nch * KCH * NW for nch in PART_NCH)
NPARTS = len(PART_NCH)


def _sc_gather(nch, ta, tb, row, col, cx, cy, cz):
    return _sc_kernels(nch * KCH, nch)[0](ta, tb, row, col, cx, cy, cz)


def _sc_scatter_m(nch, m, row3, zrows):
    return _sc_kernels(nch * KCH, nch)[1](m, row3, zrows)


def _sc_scatter_t(nch, s16, row3):
    return _sc_kernels(nch * KCH, nch)[2](s16, row3)


BN = 2000
BE = 2560


def _full(shape):
    return pl.BlockSpec(shape, lambda i: (0,) * len(shape))


def _blk(shape, pos=0):
    def imap(i):
        out = [0] * len(shape)
        out[pos] = i
        return tuple(out)
    return pl.BlockSpec(shape, imap)


def _tc_node0(h, cp, wemb, bemb, w1a, be1, w1b):
    return pl.pallas_call(
        _node0_body,
        grid=(N // BN,),
        in_specs=[_blk((BN, HID)), _blk((BN, TAIL)), _full((HID, HID)),
                  _full((1, HID)), _full((HID, HID)), _full((1, HID)),
                  _full((HID, HID))],
        out_specs=[_blk((BN, HID)), _blk((BN, HID)), _blk((BN, HID))],
        out_shape=[jax.ShapeDtypeStruct((N, HID), F32),
                   jax.ShapeDtypeStruct((N, HID), F32),
                   jax.ShapeDtypeStruct((N, HID), F32)],
    )(h, cp, wemb, bemb, w1a, be1, w1b)


def _tc_node(h, cp, vp, pms, pt, wv1, bv1, wv2, bv2, wn1a, wn1b,
             bn1, wn2, bn2, w1a, be1, w1b):
    return pl.pallas_call(
        _node_body,
        grid=(N // BN,),
        in_specs=[_blk((BN, HID)), _blk((BN, TAIL)), _blk((BN, TAIL))]
                 + [_blk((BN, HID))] * 8
                 + [_blk((BN, NT)),
                    _full((HID, HID)), _full((1, HID)), _full((HID, 1)),
                    _full((1, 1)),
                    _full((HID, HID)), _full((HID, HID)), _full((1, HID)),
                    _full((HID, HID)), _full((1, HID)),
                    _full((HID, HID)), _full((1, HID)), _full((HID, HID))],
        out_specs=[_blk((BN, HID)), _blk((BN, TAIL)), _blk((BN, HID)),
                   _blk((BN, HID))],
        out_shape=[jax.ShapeDtypeStruct((N, HID), F32),
                   jax.ShapeDtypeStruct((N, TAIL), F32),
                   jax.ShapeDtypeStruct((N, HID), F32),
                   jax.ShapeDtypeStruct((N, HID), F32)],
    )(h, cp, vp, *pms, pt, wv1, bv1, wv2, bv2, wn1a, wn1b, bn1,
      wn2, bn2, w1a, be1, w1b)


def _tc_node_last(h, cp, vp, pt, wv1, bv1, wv2, bv2):
    return pl.pallas_call(
        _node_last_body,
        grid=(N // BN,),
        in_specs=[_blk((BN, HID)), _blk((BN, TAIL)), _blk((BN, TAIL)),
                  _blk((BN, NT)),
                  _full((HID, HID)), _full((1, HID)), _full((HID, 1)),
                  _full((1, 1))],
        out_specs=_blk((BN, TAIL)),
        out_shape=jax.ShapeDtypeStruct((N, TAIL), F32),
    )(h, cp, vp, pt, wv1, bv1, wv2, bv2)


def _tc_reduce(pts):
    return pl.pallas_call(
        _reduce_body,
        grid=(1,),
        in_specs=[_full((NW, TR, HID))] * NPARTS,
        out_specs=_full((TR, HID)),
        out_shape=jax.ShapeDtypeStruct((TR, HID), F32),
    )(*pts)


def _tc_edge(g, es, ea, wr, w1e, we2, be2, wc1, bc1, wc2, want_m=True):
    eh = g.shape[0]
    body = _edge_body if want_m else _edge_body_nom
    out_specs = [_blk((BE, HID)), _blk((BE, TAIL))]
    out_shape = [jax.ShapeDtypeStruct((eh, HID), F32),
                 jax.ShapeDtypeStruct((eh, TAIL), F32)]
    if not want_m:
        out_specs, out_shape = out_specs[1:], out_shape[1:]
    res = pl.pallas_call(
        body,
        grid=(eh // BE,),
        in_specs=[_blk((BE, HID)), _blk((BE, TAIL)),
                  _blk((BE, 16)),
                  _full((1, HID)), _full((16, HID)), _full((HID, HID)),
                  _full((1, HID)), _full((HID, HID)), _full((1, HID)),
                  _full((HID, 1))],
        out_specs=out_specs,
        out_shape=out_shape,
    )(g, es, ea, wr, w1e, we2, be2, wc1, bc1, wc2)
    return res if want_m else res[0]


def kernel(h, x, edges, vel, edge_attr, params):
    row = edges[0]
    col = edges[1]
    rows, cols, row3s, eas = [], [], [], []
    off = 0
    for pi, nch in enumerate(PART_NCH):
        eh = PART_E[pi]
        rows.append(row[off:off + eh])
        cols.append(col[off:off + eh])
        row3s.append(rows[-1].reshape(NW, nch, KCH))
        eas.append(edge_attr[off:off + eh])
        off += eh
    cp = jnp.pad(x, ((0, 0), (0, TAIL - 3)))
    vp = jnp.pad(vel, ((0, 0), (0, TAIL - 3)))
    zrows = jnp.zeros((SPT, HID), F32)
    r2 = lambda b: b.reshape(1, -1)

    lp = params["layers"]
    p0w = lp[0]
    hh, ta, tb = _tc_node0(
        h, cp, params["emb"]["W"], r2(params["emb"]["b"]),
        p0w["We1"][:HID], r2(p0w["be1"]), p0w["We1"][HID:2 * HID])

    for li in range(4):
        p = lp[li]
        ew = (r2(p["We1"][2 * HID]), p["We1"][2 * HID + 1:], p["We2"],
              r2(p["be2"]), p["Wc1"], r2(p["bc1"]), p["Wc2"])
        cx, cy, cz = cp[:, 0], cp[:, 1], cp[:, 2]
        last = li == 3
        gs = [_sc_gather(nch, ta, tb, rows[pi], cols[pi], cx, cy, cz)
              for pi, nch in enumerate(PART_NCH)]
        pms, pts = [], []
        for pi, nch in enumerate(PART_NCH):
            g, es = gs[pi]
            if not last:
                m, s16 = _tc_edge(g, es, eas[pi], *ew)
                pm = _sc_scatter_m(nch, m, row3s[pi], zrows)
                pms.extend([pm[0, :N], pm[1, :N]])
            else:
                s16 = _tc_edge(g, es, eas[pi], *ew, want_m=False)
            pts.append(_sc_scatter_t(nch, s16, row3s[pi]))
        ptr = _tc_reduce(pts).reshape(TR * HID)[:N * NT].reshape(N, NT)
        if not last:
            nx = lp[li + 1]
            hh, cp, ta, tb = _tc_node(
                hh, cp, vp, pms, ptr,
                p["Wv1"], r2(p["bv1"]), p["Wv2"], r2(p["bv2"]),
                p["Wn1"][:HID], p["Wn1"][HID:], r2(p["bn1"]),
                p["Wn2"], r2(p["bn2"]),
                nx["We1"][:HID], r2(nx["be1"]), nx["We1"][HID:2 * HID])
        else:
            cp = _tc_node_last(hh, cp, vp, ptr,
                               p["Wv1"], r2(p["bv1"]), p["Wv2"],
                               r2(p["bv2"]))
    return cp[:, :3]


# speedup vs baseline: 1.0421x; 1.0421x over previous
import functools

import jax
import jax.numpy as jnp
from jax import lax
from jax.experimental import pallas as pl
from jax.experimental.pallas import tpu as pltpu
from jax.experimental.pallas import tpu_sc as plsc

N = 10000
E = 320000
HID = 128
TAIL = 16

NC, NS = 2, 16
NW = NC * NS
EPW = E // NW
KCH = 80
NCH = EPW // KCH
NP = 10240
SPT = NP // NS
NT = 4
TR = 320

F32 = jnp.float32
BF16 = jnp.bfloat16


def _silu(x):
    return x * (1.0 / (1.0 + jnp.exp(-x)))


def _node0_body(h, cp, wemb, bemb, w1a, be1, w1b, hh_o, ta_o, tb_o):
    hh = jnp.dot(h[...], wemb[...], preferred_element_type=F32) + bemb[...]
    hh_o[...] = hh
    ta_o[...] = jnp.dot(hh, w1a[...], preferred_element_type=F32) + be1[...]
    tb_o[...] = jnp.dot(hh, w1b[...], preferred_element_type=F32)


def _node_body(h, cp, vp, *rest):
    nmp = len(rest) - 17
    pms = rest[:nmp]
    (pt, wv1, bv1, wv2, bv2, wn1a, wn1b, bn1, wn2, bn2, w1a, be1, w1b,
     hn_o, cn_o, ta_o, tb_o) = rest[nmp:]
    hv = h[...]
    aggm = pms[0][...]
    for pm in pms[1:]:
        aggm = aggm + pm[...]
    tl = pt[...]
    cnt = jnp.maximum(tl[:, 3:4], 1.0)
    lane = lax.broadcasted_iota(jnp.int32, (1, TAIL), 1)
    mask3 = (lane < 3).astype(F32)
    tl16 = jnp.concatenate(
        [tl, jnp.zeros((tl.shape[0], TAIL - NT), F32)], axis=1)
    sv = (jnp.dot(_silu(jnp.dot(hv, wv1[...], preferred_element_type=F32)
                        + bv1[...]), wv2[...], preferred_element_type=F32)
          + bv2[...])
    cn = cp[...] + (tl16 * mask3) / cnt + sv * vp[...]
    cn_o[...] = cn
    t = _silu(jnp.dot(hv, wn1a[...], preferred_element_type=F32)
              + jnp.dot(aggm, wn1b[...], preferred_element_type=F32)
              + bn1[...])
    hn = hv + jnp.dot(t, wn2[...], preferred_element_type=F32) + bn2[...]
    hn_o[...] = hn
    ta_o[...] = jnp.dot(hn, w1a[...], preferred_element_type=F32) + be1[...]
    tb_o[...] = jnp.dot(hn, w1b[...], preferred_element_type=F32)


def _node_last_body(h, cp, vp, pt, wv1, bv1, wv2, bv2, cn_o):
    hv = h[...]
    tl = pt[...]
    cnt = jnp.maximum(tl[:, 3:4], 1.0)
    lane = lax.broadcasted_iota(jnp.int32, (1, TAIL), 1)
    mask3 = (lane < 3).astype(F32)
    tl16 = jnp.concatenate(
        [tl, jnp.zeros((tl.shape[0], TAIL - NT), F32)], axis=1)
    sv = (jnp.dot(_silu(jnp.dot(hv, wv1[...], preferred_element_type=F32)
                        + bv1[...]), wv2[...], preferred_element_type=F32)
          + bv2[...])
    cn_o[...] = cp[...] + (tl16 * mask3) / cnt + sv * vp[...]


def _edge_body(g, es, ea, wr, w1e, we2, be2, wc1, bc1, wc2, m_o, s_o):
    radial = es[:, 0:1]
    e1 = (g[...] + radial * wr[...]
          + jnp.dot(ea[...], w1e[...], preferred_element_type=F32))
    m = _silu(jnp.dot(_silu(e1).astype(BF16), we2[...].astype(BF16),
                      preferred_element_type=F32) + be2[...])
    m_o[...] = m
    cm = _silu(jnp.dot(m.astype(BF16), wc1[...].astype(BF16),
                       preferred_element_type=F32) + bc1[...])
    s = jnp.dot(cm, wc2[...], preferred_element_type=F32)
    s_o[...] = jnp.concatenate(
        [s, es[:, 1:4], jnp.zeros((s.shape[0], TAIL - 4), F32)], axis=1)


def _edge_body_nom(g, es, ea, wr, w1e, we2, be2, wc1, bc1, wc2, s_o):
    radial = es[:, 0:1]
    e1 = (g[...] + radial * wr[...]
          + jnp.dot(ea[...], w1e[...], preferred_element_type=F32))
    m = _silu(jnp.dot(_silu(e1).astype(BF16), we2[...].astype(BF16),
                      preferred_element_type=F32) + be2[...])
    cm = _silu(jnp.dot(m.astype(BF16), wc1[...].astype(BF16),
                       preferred_element_type=F32) + bc1[...])
    s = jnp.dot(cm, wc2[...], preferred_element_type=F32)
    s_o[...] = jnp.concatenate(
        [s, es[:, 1:4], jnp.zeros((s.shape[0], TAIL - 4), F32)], axis=1)


def _reduce_body(*refs):
    acc = jnp.sum(refs[0][...], axis=0)
    for r in refs[1:-1]:
        acc = acc + jnp.sum(r[...], axis=0)
    refs[-1][...] = acc


def _make_gather_body(epw, nch):
    pairs = (nch - 1) // 2
    odd = nch % 2 == 1

    def body_fn(ta, tb, row, col, cx, cy, cz, g_o, es_o,
                rowi, coli, bufa0, bufb0, bufa1, bufb1, gsum, esb,
                cxv, cyv, czv, sga0, sgb0, sga1, sgb1, semw0, semw1):
        wid = lax.axis_index("s") * NC + lax.axis_index("c")
        base = wid * epw
        pltpu.sync_copy(row.at[pl.ds(base, epw)], rowi)
        pltpu.sync_copy(col.at[pl.ds(base, epw)], coli)
        pltpu.sync_copy(cx, cxv)
        pltpu.sync_copy(cy, cyv)
        pltpu.sync_copy(cz, czv)
        iota = lax.iota(jnp.int32, 16)

        def issue_g(c, bufa, bufb, sa, sb):
            pltpu.async_copy(ta.at[rowi.at[pl.ds(c * KCH, KCH)]], bufa, sa)
            pltpu.async_copy(tb.at[coli.at[pl.ds(c * KCH, KCH)]], bufb, sb)

        def wait_g(c, bufa, bufb, sa, sb):
            pltpu.make_async_copy(ta.at[rowi.at[pl.ds(c * KCH, KCH)]],
                                  bufa, sa).wait()
            pltpu.make_async_copy(tb.at[coli.at[pl.ds(c * KCH, KCH)]],
                                  bufb, sb).wait()

        def compute_es(c):
            for j in range(KCH // 16):
                rv = rowi[pl.ds(c * KCH + j * 16, 16)]
                cv = coli[pl.ds(c * KCH + j * 16, 16)]
                dx = (plsc.load_gather(cxv, [rv])
                      - plsc.load_gather(cxv, [cv]))
                dy = (plsc.load_gather(cyv, [rv])
                      - plsc.load_gather(cyv, [cv]))
                dz = (plsc.load_gather(czv, [rv])
                      - plsc.load_gather(czv, [cv]))
                r = dx * dx + dy * dy + dz * dz
                ri = iota + (j * 16)
                plsc.store_scatter(esb, [ri, iota * 0], r)
                plsc.store_scatter(esb, [ri, iota * 0 + 1], dx)
                plsc.store_scatter(esb, [ri, iota * 0 + 2], dy)
                plsc.store_scatter(esb, [ri, iota * 0 + 3], dz)

        def add_rows(bufa, bufb):
            def radd(r, _):
                for j in range(HID // 16):
                    gsum[r, pl.ds(j * 16, 16)] = (
                        bufa[r, pl.ds(j * 16, 16)]
                        + bufb[r, pl.ds(j * 16, 16)])
                return 0

            lax.fori_loop(0, KCH, radd, 0)

        def drain(c, semw):
            off = base + c * KCH
            wg = pltpu.async_copy(gsum, g_o.at[pl.ds(off, KCH)], semw)
            we = pltpu.async_copy(esb, es_o.at[pl.ds(off, KCH)], semw)
            wg.wait()
            we.wait()

        def step(c, bufa, bufb, sa, sb, semw):
            compute_es(c)
            wait_g(c, bufa, bufb, sa, sb)
            add_rows(bufa, bufb)
            drain(c, semw)

        issue_g(0, bufa0, bufb0, sga0, sgb0)
        issue_g(1, bufa1, bufb1, sga1, sgb1)

        def body(i, _):
            c0 = 2 * i
            c1 = c0 + 1
            step(c0, bufa0, bufb0, sga0, sgb0, semw0)
            issue_g(c0 + 2, bufa0, bufb0, sga0, sgb0)
            step(c1, bufa1, bufb1, sga1, sgb1, semw1)
            if odd:
                @pl.when(i < pairs - 1)
                def _():
                    issue_g(c1 + 2, bufa1, bufb1, sga1, sgb1)
            else:
                issue_g(c1 + 2, bufa1, bufb1, sga1, sgb1)
            return 0

        lax.fori_loop(0, pairs, body, 0)
        if odd:
            step(nch - 1, bufa0, bufb0, sga0, sgb0, semw0)
        else:
            step(nch - 2, bufa0, bufb0, sga0, sgb0, semw0)
            step(nch - 1, bufa1, bufb1, sga1, sgb1, semw1)

    return body_fn


def _make_scatter_m_body(epw, nch):
    pairs = (nch - 1) // 2
    odd = nch % 2 == 1

    def body_fn(m, row3, zrows, pm_o, idr, mb0, mb1, accm,
                sl0, sl1, sa0, sa1):
        cid = lax.axis_index("c")
        sid = lax.axis_index("s")
        wid = sid * NC + cid
        base = wid * epw

        def issue_l(c, mb, sl):
            pltpu.async_copy(m.at[pl.ds(base + c * KCH, KCH)], mb, sl)

        def wait_l(c, mb, sl):
            pltpu.make_async_copy(m.at[pl.ds(base + c * KCH, KCH)], mb,
                                  sl).wait()

        issue_l(0, mb0, sl0)
        issue_l(1, mb1, sl1)
        pltpu.sync_copy(row3.at[wid], idr)
        pltpu.sync_copy(zrows, accm.at[pl.ds(sid * SPT, SPT)])
        plsc.subcore_barrier()

        def sadd(c, mb, sa):
            pltpu.async_copy(mb, accm.at[idr.at[c]], sa, add=True).wait()

        def step(c, mb, sl, sa):
            wait_l(c, mb, sl)
            sadd(c, mb, sa)

        def body(i, _):
            c0 = 2 * i
            c1 = c0 + 1
            step(c0, mb0, sl0, sa0)
            issue_l(c0 + 2, mb0, sl0)
            step(c1, mb1, sl1, sa1)
            if odd:
                @pl.when(i < pairs - 1)
                def _():
                    issue_l(c1 + 2, mb1, sl1)
            else:
                issue_l(c1 + 2, mb1, sl1)
            return 0

        lax.fori_loop(0, pairs, body, 0)
        if odd:
            step(nch - 1, mb0, sl0, sa0)
        else:
            step(nch - 2, mb0, sl0, sa0)
            step(nch - 1, mb1, sl1, sa1)
        plsc.subcore_barrier()
        pltpu.sync_copy(accm.at[pl.ds(sid * SPT, SPT)],
                        pm_o.at[cid, pl.ds(sid * SPT, SPT)])

    return body_fn


def _make_scatter_t_body(epw, nch):
    pairs = (nch - 1) // 2
    odd = nch % 2 == 1

    def body_fn(s16, row3, pt_o, idr, sb0, sb1, acct, ss0, ss1):
        cid = lax.axis_index("c")
        sid = lax.axis_index("s")
        wid = sid * NC + cid
        base = wid * epw

        def issue_l(c, sb, ss):
            pltpu.async_copy(s16.at[pl.ds(base + c * KCH, KCH)], sb, ss)

        def wait_l(c, sb, ss):
            pltpu.make_async_copy(s16.at[pl.ds(base + c * KCH, KCH)], sb,
                                  ss).wait()

        issue_l(0, sb0, ss0)
        issue_l(1, sb1, ss1)
        pltpu.sync_copy(row3.at[wid], idr)
        zero16 = jnp.zeros((16,), F32)

        def zbody(i, _):
            for j in range(HID // 16):
                acct[i, pl.ds(j * 16, 16)] = zero16
            return 0

        lax.fori_loop(0, TR, zbody, 0)
        iota = lax.iota(jnp.int32, 16)
        one16 = zero16 + 1.0

        def process(c, sb):
            for j in range(KCH // 16):
                rv = idr[c, pl.ds(j * 16, 16)]
                ri = iota + (j * 16)
                sv = plsc.load_gather(sb, [ri, iota * 0])
                dx = plsc.load_gather(sb, [ri, iota * 0 + 1])
                dy = plsc.load_gather(sb, [ri, iota * 0 + 2])
                dz = plsc.load_gather(sb, [ri, iota * 0 + 3])
                fb = rv * NT
                for k, v in ((0, dx * sv), (1, dy * sv), (2, dz * sv),
                             (3, one16)):
                    fk = fb + k
                    plsc.addupdate_scatter(
                        acct, [lax.shift_right_logical(fk, 7), fk & 127], v)

        def step(c, sb, ss):
            wait_l(c, sb, ss)
            process(c, sb)

        def body(i, _):
            c0 = 2 * i
            c1 = c0 + 1
            step(c0, sb0, ss0)
            issue_l(c0 + 2, sb0, ss0)
            step(c1, sb1, ss1)
            if odd:
                @pl.when(i < pairs - 1)
                def _():
                    issue_l(c1 + 2, sb1, ss1)
            else:
                issue_l(c1 + 2, sb1, ss1)
            return 0

        lax.fori_loop(0, pairs, body, 0)
        if odd:
            step(nch - 1, sb0, ss0)
        else:
            step(nch - 2, sb0, ss0)
            step(nch - 1, sb1, ss1)
        pltpu.sync_copy(acct, pt_o.at[wid])

    return body_fn


@functools.cache
def _sc_kernels(epw, nch):
    eh = epw * NW
    mesh = plsc.VectorSubcoreMesh(core_axis_name="c", subcore_axis_name="s",
                                  num_cores=NC, num_subcores=NS)
    cparams = pltpu.CompilerParams(needs_layout_passes=False)
    gather = functools.partial(
        pl.kernel,
        compiler_params=cparams,
        out_type=(jax.ShapeDtypeStruct((eh, HID), F32),
                  jax.ShapeDtypeStruct((eh, TAIL), F32)),
        mesh=mesh,
        scratch_types=[
            pltpu.VMEM((epw,), jnp.int32),
            pltpu.VMEM((epw,), jnp.int32),
            pltpu.VMEM((KCH, HID), F32),
            pltpu.VMEM((KCH, HID), F32),
            pltpu.VMEM((KCH, HID), F32),
            pltpu.VMEM((KCH, HID), F32),
            pltpu.VMEM((KCH, HID), F32),
            pltpu.VMEM((KCH, TAIL), F32),
            pltpu.VMEM((N,), F32),
            pltpu.VMEM((N,), F32),
            pltpu.VMEM((N,), F32),
            pltpu.SemaphoreType.DMA,
            pltpu.SemaphoreType.DMA,
            pltpu.SemaphoreType.DMA,
            pltpu.SemaphoreType.DMA,
            pltpu.SemaphoreType.DMA,
            pltpu.SemaphoreType.DMA,
        ],
    )(_make_gather_body(epw, nch))
    scatter_m = functools.partial(
        pl.kernel,
        compiler_params=cparams,
        out_type=jax.ShapeDtypeStruct((NC, NP, HID), F32),
        mesh=mesh,
        scratch_types=[
            pltpu.VMEM((nch, KCH), jnp.int32),
            pltpu.VMEM((KCH, HID), F32),
            pltpu.VMEM((KCH, HID), F32),
            pltpu.VMEM_SHARED((NP, HID), F32),
            pltpu.SemaphoreType.DMA,
            pltpu.SemaphoreType.DMA,
            pltpu.SemaphoreType.DMA,
            pltpu.SemaphoreType.DMA,
        ],
    )(_make_scatter_m_body(epw, nch))
    scatter_t = functools.partial(
        pl.kernel,
        compiler_params=cparams,
        out_type=jax.ShapeDtypeStruct((NW, TR, HID), F32),
        mesh=mesh,
        scratch_types=[
            pltpu.VMEM((nch, KCH), jnp.int32),
            pltpu.VMEM((KCH, TAIL), F32),
            pltpu.VMEM((KCH, TAIL), F32),
            pltpu.VMEM((TR, HID), F32),
            pltpu.SemaphoreType.DMA,
            pltpu.SemaphoreType.DMA,
        ],
    )(_make_scatter_t_body(epw, nch))
    return gather, scatter_m, scatter_t


PART_NCH = (63, 62)
PART_E = tuple(nch * KCH * NW for nch in PART_NCH)
NPARTS = len(PART_NCH)


def _sc_gather(nch, ta, tb, row, col, cx, cy, cz):
    return _sc_kernels(nch * KCH, nch)[0](ta, tb, row, col, cx, cy, cz)


def _sc_scatter_m(nch, m, row3, zrows):
    return _sc_kernels(nch * KCH, nch)[1](m, row3, zrows)


def _sc_scatter_t(nch, s16, row3):
    return _sc_kernels(nch * KCH, nch)[2](s16, row3)


BN = 2000
BE = 2560


def _full(shape):
    return pl.BlockSpec(shape, lambda i: (0,) * len(shape))


def _blk(shape, pos=0):
    def imap(i):
        out = [0] * len(shape)
        out[pos] = i
        return tuple(out)
    return pl.BlockSpec(shape, imap)


def _tc_node0(h, cp, wemb, bemb, w1a, be1, w1b):
    return pl.pallas_call(
        _node0_body,
        grid=(N // BN,),
        in_specs=[_blk((BN, HID)), _blk((BN, TAIL)), _full((HID, HID)),
                  _full((1, HID)), _full((HID, HID)), _full((1, HID)),
                  _full((HID, HID))],
        out_specs=[_blk((BN, HID)), _blk((BN, HID)), _blk((BN, HID))],
        out_shape=[jax.ShapeDtypeStruct((N, HID), F32),
                   jax.ShapeDtypeStruct((N, HID), F32),
                   jax.ShapeDtypeStruct((N, HID), F32)],
    )(h, cp, wemb, bemb, w1a, be1, w1b)


def _tc_node(h, cp, vp, pms, pt, wv1, bv1, wv2, bv2, wn1a, wn1b,
             bn1, wn2, bn2, w1a, be1, w1b):
    return pl.pallas_call(
        _node_body,
        grid=(N // BN,),
        in_specs=[_blk((BN, HID)), _blk((BN, TAIL)), _blk((BN, TAIL))]
                 + [_blk((BN, HID))] * (2 * NPARTS)
                 + [_blk((BN, NT)),
                    _full((HID, HID)), _full((1, HID)), _full((HID, 1)),
                    _full((1, 1)),
                    _full((HID, HID)), _full((HID, HID)), _full((1, HID)),
                    _full((HID, HID)), _full((1, HID)),
                    _full((HID, HID)), _full((1, HID)), _full((HID, HID))],
        out_specs=[_blk((BN, HID)), _blk((BN, TAIL)), _blk((BN, HID)),
                   _blk((BN, HID))],
        out_shape=[jax.ShapeDtypeStruct((N, HID), F32),
                   jax.ShapeDtypeStruct((N, TAIL), F32),
                   jax.ShapeDtypeStruct((N, HID), F32),
                   jax.ShapeDtypeStruct((N, HID), F32)],
    )(h, cp, vp, *pms, pt, wv1, bv1, wv2, bv2, wn1a, wn1b, bn1,
      wn2, bn2, w1a, be1, w1b)


def _tc_node_last(h, cp, vp, pt, wv1, bv1, wv2, bv2):
    return pl.pallas_call(
        _node_last_body,
        grid=(N // BN,),
        in_specs=[_blk((BN, HID)), _blk((BN, TAIL)), _blk((BN, TAIL)),
                  _blk((BN, NT)),
                  _full((HID, HID)), _full((1, HID)), _full((HID, 1)),
                  _full((1, 1))],
        out_specs=_blk((BN, TAIL)),
        out_shape=jax.ShapeDtypeStruct((N, TAIL), F32),
    )(h, cp, vp, pt, wv1, bv1, wv2, bv2)


def _tc_reduce(pts):
    return pl.pallas_call(
        _reduce_body,
        grid=(1,),
        in_specs=[_full((NW, TR, HID))] * NPARTS,
        out_specs=_full((TR, HID)),
        out_shape=jax.ShapeDtypeStruct((TR, HID), F32),
    )(*pts)


def _tc_edge(g, es, ea, wr, w1e, we2, be2, wc1, bc1, wc2, want_m=True):
    eh = g.shape[0]
    body = _edge_body if want_m else _edge_body_nom
    out_specs = [_blk((BE, HID)), _blk((BE, TAIL))]
    out_shape = [jax.ShapeDtypeStruct((eh, HID), F32),
                 jax.ShapeDtypeStruct((eh, TAIL), F32)]
    if not want_m:
        out_specs, out_shape = out_specs[1:], out_shape[1:]
    res = pl.pallas_call(
        body,
        grid=(eh // BE,),
        in_specs=[_blk((BE, HID)), _blk((BE, TAIL)),
                  _blk((BE, 16)),
                  _full((1, HID)), _full((16, HID)), _full((HID, HID)),
                  _full((1, HID)), _full((HID, HID)), _full((1, HID)),
                  _full((HID, 1))],
        out_specs=out_specs,
        out_shape=out_shape,
    )(g, es, ea, wr, w1e, we2, be2, wc1, bc1, wc2)
    return res if want_m else res[0]


def kernel(h, x, edges, vel, edge_attr, params):
    row = edges[0]
    col = edges[1]
    rows, cols, row3s, eas = [], [], [], []
    off = 0
    for pi, nch in enumerate(PART_NCH):
        eh = PART_E[pi]
        rows.append(row[off:off + eh])
        cols.append(col[off:off + eh])
        row3s.append(rows[-1].reshape(NW, nch, KCH))
        eas.append(edge_attr[off:off + eh])
        off += eh
    cp = jnp.pad(x, ((0, 0), (0, TAIL - 3)))
    vp = jnp.pad(vel, ((0, 0), (0, TAIL - 3)))
    zrows = jnp.zeros((SPT, HID), F32)
    r2 = lambda b: b.reshape(1, -1)

    lp = params["layers"]
    p0w = lp[0]
    hh, ta, tb = _tc_node0(
        h, cp, params["emb"]["W"], r2(params["emb"]["b"]),
        p0w["We1"][:HID], r2(p0w["be1"]), p0w["We1"][HID:2 * HID])

    for li in range(4):
        p = lp[li]
        ew = (r2(p["We1"][2 * HID]), p["We1"][2 * HID + 1:], p["We2"],
              r2(p["be2"]), p["Wc1"], r2(p["bc1"]), p["Wc2"])
        cx, cy, cz = cp[:, 0], cp[:, 1], cp[:, 2]
        last = li == 3
        gs = [_sc_gather(nch, ta, tb, rows[pi], cols[pi], cx, cy, cz)
              for pi, nch in enumerate(PART_NCH)]
        pms, pts = [], []
        for pi, nch in enumerate(PART_NCH):
            g, es = gs[pi]
            if not last:
                m, s16 = _tc_edge(g, es, eas[pi], *ew)
                pm = _sc_scatter_m(nch, m, row3s[pi], zrows)
                pms.extend([pm[0, :N], pm[1, :N]])
            else:
                s16 = _tc_edge(g, es, eas[pi], *ew, want_m=False)
            pts.append(_sc_scatter_t(nch, s16, row3s[pi]))
        ptr = _tc_reduce(pts).reshape(TR * HID)[:N * NT].reshape(N, NT)
        if not last:
            nx = lp[li + 1]
            hh, cp, ta, tb = _tc_node(
                hh, cp, vp, pms, ptr,
                p["Wv1"], r2(p["bv1"]), p["Wv2"], r2(p["bv2"]),
                p["Wn1"][:HID], p["Wn1"][HID:], r2(p["bn1"]),
                p["Wn2"], r2(p["bn2"]),
                nx["We1"][:HID], r2(nx["be1"]), nx["We1"][HID:2 * HID])
        else:
            cp = _tc_node_last(hh, cp, vp, ptr,
                               p["Wv1"], r2(p["bv1"]), p["Wv2"],
                               r2(p["bv2"]))
    return cp[:, :3]
